# Initial kernel scaffold; baseline (speedup 1.0000x reference)
import jax
import jax.numpy as jnp
from jax.experimental import pallas as pl


def kernel(x, edge_index, batch, W_feat, b_feat, W1, b1, W2, b2, W3, b3, Wfc1, bfc1, gamma, beta, Wfc2, bfc2):
    raise NotImplementedError("write your pallas kernel here")


# speedup vs baseline: 7.8419x; 7.8419x over previous
import functools

import jax
import jax.numpy as jnp
from jax import lax
from jax.experimental import pallas as pl
from jax.experimental.pallas import tpu as pltpu
from jax.experimental.pallas import tpu_sc as plsc

N = 50000
E = 800000
NG = 64
CH = 32
R = 2000
GRID = N // R
NSUB = 16
TPT = 3128
N_PAD = NSUB * TPT
EW = 125
EROWS = E // EW


def _feat_body(x_ref, w_ref, b_ref, o_ref):
    o_ref[...] = jnp.maximum(
        jnp.dot(x_ref[...], w_ref[...], preferred_element_type=jnp.float32)
        + b_ref[...], 0.0)


def _feat(x, W, b):
    return pl.pallas_call(
        _feat_body,
        grid=(GRID,),
        in_specs=[pl.BlockSpec((R, 128), lambda i: (i, 0)),
                  pl.BlockSpec((128, 64), lambda i: (0, 0)),
                  pl.BlockSpec((1, 64), lambda i: (0, 0))],
        out_specs=pl.BlockSpec((R, 64), lambda i: (i, 0)),
        out_shape=jax.ShapeDtypeStruct((N, 64), jnp.float32),
    )(x, W, b.reshape(1, 64))


def _dinv_body(a_ref, b_ref, o_ref):
    o_ref[...] = lax.rsqrt(a_ref[...] + b_ref[...] + 1.0)


def _dinv(d0, d1):
    return pl.pallas_call(
        _dinv_body,
        grid=(GRID,),
        in_specs=[pl.BlockSpec((R, 1), lambda i: (i, 0))] * 2,
        out_specs=pl.BlockSpec((R, 1), lambda i: (i, 0)),
        out_shape=jax.ShapeDtypeStruct((N, 1), jnp.float32),
    )(d0, d1)


def _conv1_body(h_ref, d_ref, w_ref, o0, o1):
    xs = d_ref[...] * jnp.dot(h_ref[...], w_ref[...],
                              preferred_element_type=jnp.float32)
    o0[...] = xs[:, 0:CH]
    o1[...] = xs[:, CH:2 * CH]


def _conv1(h0, dinv, W):
    return pl.pallas_call(
        _conv1_body,
        grid=(GRID,),
        in_specs=[pl.BlockSpec((R, 64), lambda i: (i, 0)),
                  pl.BlockSpec((R, 1), lambda i: (i, 0)),
                  pl.BlockSpec((64, 64), lambda i: (0, 0))],
        out_specs=[pl.BlockSpec((R, CH), lambda i: (i, 0))] * 2,
        out_shape=[jax.ShapeDtypeStruct((N, CH), jnp.float32)] * 2,
    )(h0, dinv, W)


def _conv_next(accs, xss, dinv, b_prev, W, h_out):
    n_in = len(accs)
    n_out = h_out // CH
    h_in = n_in * CH

    def body(*refs):
        acc_r = refs[:n_in]
        xs_r = refs[n_in:2 * n_in]
        d_ref, b_ref, w_ref = refs[2 * n_in:2 * n_in + 3]
        outs = refs[2 * n_in + 3:]
        acc = jnp.concatenate([r[...] for r in acc_r], axis=1)
        xs = jnp.concatenate([r[...] for r in xs_r], axis=1)
        h = jnp.maximum(d_ref[...] * (acc + xs) + b_ref[...], 0.0)
        xsn = d_ref[...] * jnp.dot(h, w_ref[...],
                                   preferred_element_type=jnp.float32)
        for k, o in enumerate(outs):
            o[...] = xsn[:, k * CH:(k + 1) * CH]

    return pl.pallas_call(
        body,
        grid=(GRID,),
        in_specs=([pl.BlockSpec((R, CH), lambda i: (i, 0))] * (2 * n_in)
                  + [pl.BlockSpec((R, 1), lambda i: (i, 0)),
                     pl.BlockSpec((1, h_in), lambda i: (0, 0)),
                     pl.BlockSpec((h_in, h_out), lambda i: (0, 0))]),
        out_specs=[pl.BlockSpec((R, CH), lambda i: (i, 0))] * n_out,
        out_shape=[jax.ShapeDtypeStruct((N, CH), jnp.float32)] * n_out,
    )(*accs, *xss, dinv, b_prev.reshape(1, h_in), W)


def _pool(accs, xss, dinv, b3, batch2d):
    n_in = len(accs)
    h_in = n_in * CH

    def body(*refs):
        acc_r = refs[:n_in]
        xs_r = refs[n_in:2 * n_in]
        d_ref, b_ref, g_ref = refs[2 * n_in:2 * n_in + 3]
        s_ref, c_ref = refs[2 * n_in + 3:]
        acc = jnp.concatenate([r[...] for r in acc_r], axis=1)
        xs = jnp.concatenate([r[...] for r in xs_r], axis=1)
        h = jnp.maximum(d_ref[...] * (acc + xs) + b_ref[...], 0.0)
        onehot = (g_ref[...] == lax.broadcasted_iota(jnp.int32, (1, NG), 1)
                  ).astype(jnp.float32)
        ps = lax.dot_general(onehot, h, (((0,), (0,)), ((), ())),
                             preferred_element_type=jnp.float32)
        pc = jnp.broadcast_to(jnp.sum(onehot, axis=0)[:, None], (NG, h_in))

        @pl.when(pl.program_id(0) == 0)
        def _():
            s_ref[...] = jnp.zeros_like(s_ref)
            c_ref[...] = jnp.zeros_like(c_ref)

        s_ref[...] += ps
        c_ref[...] += pc

    return pl.pallas_call(
        body,
        grid=(GRID,),
        in_specs=([pl.BlockSpec((R, CH), lambda i: (i, 0))] * (2 * n_in)
                  + [pl.BlockSpec((R, 1), lambda i: (i, 0)),
                     pl.BlockSpec((1, h_in), lambda i: (0, 0)),
                     pl.BlockSpec((R, 1), lambda i: (i, 0))]),
        out_specs=[pl.BlockSpec((NG, h_in), lambda i: (0, 0))] * 2,
        out_shape=[jax.ShapeDtypeStruct((NG, h_in), jnp.float32)] * 2,
    )(*accs, *xss, dinv, b3.reshape(1, h_in), batch2d)


def _head_body(s_ref, c_ref, w1_ref, b1_ref, g_ref, be_ref, w2_ref, b2_ref,
               o_ref):
    pooled = s_ref[...] / jnp.maximum(c_ref[...], 1.0)
    z = jnp.dot(pooled, w1_ref[...], preferred_element_type=jnp.float32) \
        + b1_ref[...]
    mu = jnp.mean(z, axis=0, keepdims=True)
    var = jnp.mean((z - mu) ** 2, axis=0, keepdims=True)
    z = (z - mu) * lax.rsqrt(var + 1e-5) * g_ref[...] + be_ref[...]
    z = jnp.maximum(z, 0.0)
    z = jnp.dot(z, w2_ref[...], preferred_element_type=jnp.float32) \
        + b2_ref[...]
    o_ref[...] = 1.0 / (1.0 + jnp.exp(-z))


def _head(sums, cnts, Wfc1, bfc1, gamma, beta, Wfc2, bfc2):
    FC = Wfc1.shape[1]
    OUT = Wfc2.shape[1]
    HI = sums.shape[1]
    return pl.pallas_call(
        _head_body,
        in_specs=[pl.BlockSpec((NG, HI), lambda: (0, 0)),
                  pl.BlockSpec((NG, HI), lambda: (0, 0)),
                  pl.BlockSpec((HI, FC), lambda: (0, 0)),
                  pl.BlockSpec((1, FC), lambda: (0, 0)),
                  pl.BlockSpec((1, FC), lambda: (0, 0)),
                  pl.BlockSpec((1, FC), lambda: (0, 0)),
                  pl.BlockSpec((FC, OUT), lambda: (0, 0)),
                  pl.BlockSpec((1, OUT), lambda: (0, 0))],
        out_specs=pl.BlockSpec((NG, OUT), lambda: (0, 0)),
        out_shape=jax.ShapeDtypeStruct((NG, OUT), jnp.float32),
    )(sums, cnts, Wfc1, bfc1.reshape(1, FC), gamma.reshape(1, FC),
      beta.reshape(1, FC), Wfc2, bfc2.reshape(1, OUT))


_MESH = dict(core_axis_name="c", subcore_axis_name="s")
DW = 32


def _deg(dst2d):
    ept_rows = EROWS // 32
    nb = ept_rows // 8

    @functools.partial(
        pl.kernel,
        out_type=jax.ShapeDtypeStruct((2, N_PAD, DW), jnp.float32),
        mesh=plsc.VectorSubcoreMesh(**_MESH),
        compiler_params=pltpu.CompilerParams(use_tc_tiling_on_sc=False),
        scratch_types=[
            pltpu.VMEM((8, EW), jnp.int32),
            pltpu.VMEM((EW, DW), jnp.float32),
            pltpu.VMEM((184, DW), jnp.float32),
            pltpu.VMEM_SHARED((N_PAD, DW), jnp.float32),
        ])
    def k(dst_hbm, o, didx, ones_v, zbuf, acc):
        c = lax.axis_index("c")
        s = lax.axis_index("s")
        wid = c * NSUB + s

        @pl.loop(0, EW)
        def _(i):
            ones_v[i, pl.ds(0, 16)] = jnp.full((16,), 1.0, jnp.float32)
            ones_v[i, pl.ds(16, 16)] = jnp.full((16,), 1.0, jnp.float32)

        @pl.loop(0, 184)
        def _(i):
            zbuf[i, pl.ds(0, 16)] = jnp.zeros((16,), jnp.float32)
            zbuf[i, pl.ds(16, 16)] = jnp.zeros((16,), jnp.float32)

        for z in range(17):
            pltpu.sync_copy(zbuf, acc.at[pl.ds(s * TPT + z * 184, 184)])
        plsc.subcore_barrier()

        @pl.loop(0, nb)
        def _(j):
            row0 = wid * ept_rows + j * 8
            pltpu.sync_copy(dst_hbm.at[pl.ds(row0, 8)], didx)
            for r in range(8):
                pltpu.sync_copy(ones_v, acc.at[didx.at[r]], add=True)

        plsc.subcore_barrier()
        pltpu.sync_copy(acc.at[pl.ds(s * TPT, TPT)],
                        o.at[c, pl.ds(s * TPT, TPT)])

    return k(dst2d)


def _agg(src2d, dst2d, xs_chunks):
    n_chunks = len(xs_chunks)
    rows_pt = EROWS // NSUB
    nb = rows_pt // 8

    @functools.partial(
        pl.kernel,
        out_type=[jax.ShapeDtypeStruct((N_PAD, CH), jnp.float32)] * n_chunks,
        mesh=plsc.VectorSubcoreMesh(**_MESH),
        compiler_params=pltpu.CompilerParams(use_tc_tiling_on_sc=False),
        scratch_types=[
            pltpu.VMEM((8, EW), jnp.int32),
            pltpu.VMEM((8, EW), jnp.int32),
            pltpu.VMEM((EW, CH), jnp.float32),
            pltpu.VMEM((184, CH), jnp.float32),
            pltpu.VMEM_SHARED((N_PAD, CH), jnp.float32),
            pltpu.SemaphoreType.DMA,
        ])
    def k(src_hbm, dst_hbm, *rest):
        xs_refs = rest[:n_chunks]
        out_refs = rest[n_chunks:2 * n_chunks]
        sidx, didx, rows, zbuf, acc, sem = rest[2 * n_chunks:]
        c = lax.axis_index("c")
        s = lax.axis_index("s")

        @pl.loop(0, 184)
        def _(i):
            zbuf[i, pl.ds(0, 16)] = jnp.zeros((16,), jnp.float32)
            zbuf[i, pl.ds(16, 16)] = jnp.zeros((16,), jnp.float32)

        for ch in range(n_chunks):
            @pl.when(c == (ch % 2))
            def _(ch=ch):
                xs_h = xs_refs[ch]
                o_h = out_refs[ch]
                for z in range(17):
                    pltpu.sync_copy(zbuf,
                                    acc.at[pl.ds(s * TPT + z * 184, 184)])
                plsc.subcore_barrier()

                @pl.loop(0, nb)
                def _(j):
                    row0 = s * rows_pt + j * 8
                    pltpu.sync_copy(src_hbm.at[pl.ds(row0, 8)], sidx)
                    pltpu.sync_copy(dst_hbm.at[pl.ds(row0, 8)], didx)
                    for r in range(8):
                        pltpu.async_copy(xs_h.at[sidx.at[r]], rows,
                                         sem).wait()
                        pltpu.sync_copy(rows, acc.at[didx.at[r]], add=True)

                plsc.subcore_barrier()
                pltpu.sync_copy(acc.at[pl.ds(s * TPT, TPT)],
                                o_h.at[pl.ds(s * TPT, TPT)])

    return k(src2d, dst2d, *xs_chunks)


def kernel(x, edge_index, batch, W_feat, b_feat, W1, b1, W2, b2, W3, b3,
           Wfc1, bfc1, gamma, beta, Wfc2, bfc2):
    src2d = edge_index[0].reshape(EROWS, EW)
    dst2d = edge_index[1].reshape(EROWS, EW)
    batch2d = batch.reshape(N, 1)

    h0 = _feat(x, W_feat, b_feat)
    d = _deg(dst2d)
    dinv = _dinv(d[0, :N, :1], d[1, :N, :1])

    xs1 = _conv1(h0, dinv, W1)
    acc1 = _agg(src2d, dst2d, xs1)
    xs2 = _conv_next(acc1, xs1, dinv, b1, W2, 128)
    acc2 = _agg(src2d, dst2d, xs2)
    xs3 = _conv_next(acc2, xs2, dinv, b2, W3, 256)
    acc3 = _agg(src2d, dst2d, xs3)
    sums, cnts = _pool(acc3, xs3, dinv, b3, batch2d)
    return _head(sums, cnts, Wfc1, bfc1, gamma, beta, Wfc2, bfc2)


# speedup vs baseline: 13.1246x; 1.6736x over previous
import functools

import jax
import jax.numpy as jnp
from jax import lax
from jax.experimental import pallas as pl
from jax.experimental.pallas import tpu as pltpu
from jax.experimental.pallas import tpu_sc as plsc

N = 50000
E = 800000
NG = 64
CH = 32
R = 2000
GRID = N // R
NSUB = 16
TPT = 3128
N_PAD = NSUB * TPT
EW = 125
EROWS = E // EW


def _feat_body(x_ref, w_ref, b_ref, o_ref):
    o_ref[...] = jnp.maximum(
        jnp.dot(x_ref[...], w_ref[...], preferred_element_type=jnp.float32)
        + b_ref[...], 0.0)


def _feat(x, W, b):
    return pl.pallas_call(
        _feat_body,
        grid=(GRID,),
        in_specs=[pl.BlockSpec((R, 128), lambda i: (i, 0)),
                  pl.BlockSpec((128, 64), lambda i: (0, 0)),
                  pl.BlockSpec((1, 64), lambda i: (0, 0))],
        out_specs=pl.BlockSpec((R, 64), lambda i: (i, 0)),
        out_shape=jax.ShapeDtypeStruct((N, 64), jnp.float32),
    )(x, W, b.reshape(1, 64))


def _dinv_body(a_ref, b_ref, o_ref):
    o_ref[...] = lax.rsqrt(a_ref[...] + b_ref[...] + 1.0)


def _dinv(d0, d1):
    return pl.pallas_call(
        _dinv_body,
        grid=(GRID,),
        in_specs=[pl.BlockSpec((R, 1), lambda i: (i, 0))] * 2,
        out_specs=pl.BlockSpec((R, 1), lambda i: (i, 0)),
        out_shape=jax.ShapeDtypeStruct((N, 1), jnp.float32),
    )(d0, d1)


def _conv1_body(h_ref, d_ref, w_ref, o0, o1):
    xs = d_ref[...] * jnp.dot(h_ref[...], w_ref[...],
                              preferred_element_type=jnp.float32)
    o0[...] = xs[:, 0:CH]
    o1[...] = xs[:, CH:2 * CH]


def _conv1(h0, dinv, W):
    return pl.pallas_call(
        _conv1_body,
        grid=(GRID,),
        in_specs=[pl.BlockSpec((R, 64), lambda i: (i, 0)),
                  pl.BlockSpec((R, 1), lambda i: (i, 0)),
                  pl.BlockSpec((64, 64), lambda i: (0, 0))],
        out_specs=[pl.BlockSpec((R, CH), lambda i: (i, 0))] * 2,
        out_shape=[jax.ShapeDtypeStruct((N, CH), jnp.float32)] * 2,
    )(h0, dinv, W)


def _conv_next(accs, xss, dinv, b_prev, W, h_out):
    n_in = len(accs)
    n_out = h_out // CH
    h_in = n_in * CH

    def body(*refs):
        acc_r = refs[:n_in]
        xs_r = refs[n_in:2 * n_in]
        d_ref, b_ref, w_ref = refs[2 * n_in:2 * n_in + 3]
        outs = refs[2 * n_in + 3:]
        acc = jnp.concatenate([r[...] for r in acc_r], axis=1)
        xs = jnp.concatenate([r[...] for r in xs_r], axis=1)
        h = jnp.maximum(d_ref[...] * (acc + xs) + b_ref[...], 0.0)
        xsn = d_ref[...] * jnp.dot(h, w_ref[...],
                                   preferred_element_type=jnp.float32)
        for k, o in enumerate(outs):
            o[...] = xsn[:, k * CH:(k + 1) * CH]

    return pl.pallas_call(
        body,
        grid=(GRID,),
        in_specs=([pl.BlockSpec((R, CH), lambda i: (i, 0))] * (2 * n_in)
                  + [pl.BlockSpec((R, 1), lambda i: (i, 0)),
                     pl.BlockSpec((1, h_in), lambda i: (0, 0)),
                     pl.BlockSpec((h_in, h_out), lambda i: (0, 0))]),
        out_specs=[pl.BlockSpec((R, CH), lambda i: (i, 0))] * n_out,
        out_shape=[jax.ShapeDtypeStruct((N, CH), jnp.float32)] * n_out,
    )(*accs, *xss, dinv, b_prev.reshape(1, h_in), W)


def _pool(accs, xss, dinv, b3, batch2d):
    n_in = len(accs)
    h_in = n_in * CH

    def body(*refs):
        acc_r = refs[:n_in]
        xs_r = refs[n_in:2 * n_in]
        d_ref, b_ref, g_ref = refs[2 * n_in:2 * n_in + 3]
        s_ref, c_ref = refs[2 * n_in + 3:]
        acc = jnp.concatenate([r[...] for r in acc_r], axis=1)
        xs = jnp.concatenate([r[...] for r in xs_r], axis=1)
        h = jnp.maximum(d_ref[...] * (acc + xs) + b_ref[...], 0.0)
        onehot = (g_ref[...] == lax.broadcasted_iota(jnp.int32, (1, NG), 1)
                  ).astype(jnp.float32)
        ps = lax.dot_general(onehot, h, (((0,), (0,)), ((), ())),
                             preferred_element_type=jnp.float32)
        pc = jnp.broadcast_to(jnp.sum(onehot, axis=0)[:, None], (NG, h_in))

        @pl.when(pl.program_id(0) == 0)
        def _():
            s_ref[...] = jnp.zeros_like(s_ref)
            c_ref[...] = jnp.zeros_like(c_ref)

        s_ref[...] += ps
        c_ref[...] += pc

    return pl.pallas_call(
        body,
        grid=(GRID,),
        in_specs=([pl.BlockSpec((R, CH), lambda i: (i, 0))] * (2 * n_in)
                  + [pl.BlockSpec((R, 1), lambda i: (i, 0)),
                     pl.BlockSpec((1, h_in), lambda i: (0, 0)),
                     pl.BlockSpec((R, 1), lambda i: (i, 0))]),
        out_specs=[pl.BlockSpec((NG, h_in), lambda i: (0, 0))] * 2,
        out_shape=[jax.ShapeDtypeStruct((NG, h_in), jnp.float32)] * 2,
    )(*accs, *xss, dinv, b3.reshape(1, h_in), batch2d)


def _head_body(s_ref, c_ref, w1_ref, b1_ref, g_ref, be_ref, w2_ref, b2_ref,
               o_ref):
    pooled = s_ref[...] / jnp.maximum(c_ref[...], 1.0)
    z = jnp.dot(pooled, w1_ref[...], preferred_element_type=jnp.float32) \
        + b1_ref[...]
    mu = jnp.mean(z, axis=0, keepdims=True)
    var = jnp.mean((z - mu) ** 2, axis=0, keepdims=True)
    z = (z - mu) * lax.rsqrt(var + 1e-5) * g_ref[...] + be_ref[...]
    z = jnp.maximum(z, 0.0)
    z = jnp.dot(z, w2_ref[...], preferred_element_type=jnp.float32) \
        + b2_ref[...]
    o_ref[...] = 1.0 / (1.0 + jnp.exp(-z))


def _head(sums, cnts, Wfc1, bfc1, gamma, beta, Wfc2, bfc2):
    FC = Wfc1.shape[1]
    OUT = Wfc2.shape[1]
    HI = sums.shape[1]
    return pl.pallas_call(
        _head_body,
        in_specs=[pl.BlockSpec((NG, HI), lambda: (0, 0)),
                  pl.BlockSpec((NG, HI), lambda: (0, 0)),
                  pl.BlockSpec((HI, FC), lambda: (0, 0)),
                  pl.BlockSpec((1, FC), lambda: (0, 0)),
                  pl.BlockSpec((1, FC), lambda: (0, 0)),
                  pl.BlockSpec((1, FC), lambda: (0, 0)),
                  pl.BlockSpec((FC, OUT), lambda: (0, 0)),
                  pl.BlockSpec((1, OUT), lambda: (0, 0))],
        out_specs=pl.BlockSpec((NG, OUT), lambda: (0, 0)),
        out_shape=jax.ShapeDtypeStruct((NG, OUT), jnp.float32),
    )(sums, cnts, Wfc1, bfc1.reshape(1, FC), gamma.reshape(1, FC),
      beta.reshape(1, FC), Wfc2, bfc2.reshape(1, OUT))


_MESH = dict(core_axis_name="c", subcore_axis_name="s")
DW = 32


def _deg(dst2d):
    ept_rows = EROWS // 32
    nb = ept_rows // 8

    @functools.partial(
        pl.kernel,
        out_type=jax.ShapeDtypeStruct((2, N_PAD, DW), jnp.float32),
        mesh=plsc.VectorSubcoreMesh(**_MESH),
        compiler_params=pltpu.CompilerParams(use_tc_tiling_on_sc=False),
        scratch_types=[
            pltpu.VMEM((8, EW), jnp.int32),
            pltpu.VMEM((EW, DW), jnp.float32),
            pltpu.VMEM((184, DW), jnp.float32),
            pltpu.VMEM_SHARED((N_PAD, DW), jnp.float32),
        ])
    def k(dst_hbm, o, didx, ones_v, zbuf, acc):
        c = lax.axis_index("c")
        s = lax.axis_index("s")
        wid = c * NSUB + s

        @pl.loop(0, EW)
        def _(i):
            ones_v[i, pl.ds(0, 16)] = jnp.full((16,), 1.0, jnp.float32)
            ones_v[i, pl.ds(16, 16)] = jnp.full((16,), 1.0, jnp.float32)

        @pl.loop(0, 184)
        def _(i):
            zbuf[i, pl.ds(0, 16)] = jnp.zeros((16,), jnp.float32)
            zbuf[i, pl.ds(16, 16)] = jnp.zeros((16,), jnp.float32)

        for z in range(17):
            pltpu.sync_copy(zbuf, acc.at[pl.ds(s * TPT + z * 184, 184)])
        plsc.subcore_barrier()

        @pl.loop(0, nb)
        def _(j):
            row0 = wid * ept_rows + j * 8
            pltpu.sync_copy(dst_hbm.at[pl.ds(row0, 8)], didx)
            for r in range(8):
                pltpu.sync_copy(ones_v, acc.at[didx.at[r]], add=True)

        plsc.subcore_barrier()
        pltpu.sync_copy(acc.at[pl.ds(s * TPT, TPT)],
                        o.at[c, pl.ds(s * TPT, TPT)])

    return k(dst2d)


def _agg(src2d, dst2d, xs_chunks):
    n_chunks = len(xs_chunks)
    rows_pt = EROWS // NSUB
    nb = rows_pt // 8

    @functools.partial(
        pl.kernel,
        out_type=[jax.ShapeDtypeStruct((N_PAD, CH), jnp.float32)] * n_chunks,
        mesh=plsc.VectorSubcoreMesh(**_MESH),
        compiler_params=pltpu.CompilerParams(use_tc_tiling_on_sc=False),
        scratch_types=[
            pltpu.VMEM((2, 8, EW), jnp.int32),
            pltpu.VMEM((2, 8, EW), jnp.int32),
            [pltpu.VMEM((EW, CH), jnp.float32)] * 4,
            pltpu.VMEM((184, CH), jnp.float32),
            pltpu.VMEM_SHARED((N_PAD, CH), jnp.float32),
            [pltpu.SemaphoreType.DMA] * 4,
            [pltpu.SemaphoreType.DMA] * 4,
        ])
    def k(src_hbm, dst_hbm, *rest):
        xs_refs = rest[:n_chunks]
        out_refs = rest[n_chunks:2 * n_chunks]
        sidx, didx, rows, zbuf, acc, gsem, ssem = rest[2 * n_chunks:]
        c = lax.axis_index("c")
        s = lax.axis_index("s")

        @pl.loop(0, 184)
        def _(i):
            zbuf[i, pl.ds(0, 16)] = jnp.zeros((16,), jnp.float32)
            zbuf[i, pl.ds(16, 16)] = jnp.zeros((16,), jnp.float32)

        for ch in range(n_chunks):
            @pl.when(c == (ch % 2))
            def _(ch=ch):
                xs_h = xs_refs[ch]
                o_h = out_refs[ch]
                for z in range(17):
                    pltpu.sync_copy(zbuf,
                                    acc.at[pl.ds(s * TPT + z * 184, 184)])
                plsc.subcore_barrier()

                @pl.loop(0, nb // 2)
                def _(t):
                    for half in range(2):
                        row0 = s * rows_pt + (2 * t + half) * 8
                        pltpu.sync_copy(src_hbm.at[pl.ds(row0, 8)],
                                        sidx.at[half])
                        pltpu.sync_copy(dst_hbm.at[pl.ds(row0, 8)],
                                        didx.at[half])
                        for wave in range(2):
                            gd = [None] * 4
                            for q in range(4):
                                r = wave * 4 + q
                                drain = pltpu.make_async_copy(
                                    rows[q], acc.at[didx.at[half, r]],
                                    ssem[q])
                                if half == 0 and wave == 0:
                                    @pl.when(t > 0)
                                    def _(drain=drain):
                                        drain.wait()
                                else:
                                    drain.wait()
                                gd[q] = pltpu.async_copy(
                                    xs_h.at[sidx.at[half, r]], rows[q],
                                    gsem[q])
                            for q in range(4):
                                r = wave * 4 + q
                                gd[q].wait()
                                pltpu.async_copy(rows[q],
                                                 acc.at[didx.at[half, r]],
                                                 ssem[q], add=True)

                for q in range(4):
                    pltpu.make_async_copy(rows[q], acc.at[didx.at[1, 4 + q]],
                                          ssem[q]).wait()
                plsc.subcore_barrier()
                pltpu.sync_copy(acc.at[pl.ds(s * TPT, TPT)],
                                o_h.at[pl.ds(s * TPT, TPT)])

    return k(src2d, dst2d, *xs_chunks)


def kernel(x, edge_index, batch, W_feat, b_feat, W1, b1, W2, b2, W3, b3,
           Wfc1, bfc1, gamma, beta, Wfc2, bfc2):
    src2d = edge_index[0].reshape(EROWS, EW)
    dst2d = edge_index[1].reshape(EROWS, EW)
    batch2d = batch.reshape(N, 1)

    h0 = _feat(x, W_feat, b_feat)
    d = _deg(dst2d)
    dinv = _dinv(d[0, :N, :1], d[1, :N, :1])

    xs1 = _conv1(h0, dinv, W1)
    acc1 = _agg(src2d, dst2d, xs1)
    xs2 = _conv_next(acc1, xs1, dinv, b1, W2, 128)
    acc2 = _agg(src2d, dst2d, xs2)
    xs3 = _conv_next(acc2, xs2, dinv, b2, W3, 256)
    acc3 = _agg(src2d, dst2d, xs3)
    sums, cnts = _pool(acc3, xs3, dinv, b3, batch2d)
    return _head(sums, cnts, Wfc1, bfc1, gamma, beta, Wfc2, bfc2)
